# baseline (speedup 1.0000x reference)
import functools

import jax
import jax.numpy as jnp
from jax import lax
from jax.experimental import pallas as pl
from jax.experimental.pallas import tpu as pltpu
from jax.experimental.pallas import tpu_sc as plsc

NUM_TOKENS = 100000
EMBED_DIM = 32
BATCH = 16384
HIST = 200

B = BATCH * HIST
NW = 32
CB = 512
CH_PER_H = BATCH // CB
N_CH = B // CB
CPW = N_CH // NW
SUPER = 20
SUP_IDX = SUPER * CB
N_SUP = CPW // SUPER
TILE = 8 * 128
SEG = 4 * TILE
CHUNK_OUT = EMBED_DIM * CB
OUT_ELEMS = B * EMBED_DIM

assert CPW == N_SUP * SUPER and N_SUP % 2 == 0


def _make_kernel():
    mesh = plsc.VectorSubcoreMesh(core_axis_name="c", subcore_axis_name="s")

    @functools.partial(
        pl.kernel,
        out_type=jax.ShapeDtypeStruct((OUT_ELEMS // 128, 128), jnp.float32),
        mesh=mesh,
        compiler_params=pltpu.CompilerParams(
            use_tc_tiling_on_sc=False, needs_layout_passes=False),
        scratch_types=[
            pltpu.VMEM((SUP_IDX,), jnp.int32),
            pltpu.VMEM((SUP_IDX,), jnp.int32),
            pltpu.VMEM((CB, EMBED_DIM), jnp.float32),
            pltpu.VMEM((CB, EMBED_DIM), jnp.float32),
            pltpu.VMEM((CB, EMBED_DIM), jnp.float32),
            pltpu.VMEM((CB, EMBED_DIM), jnp.float32),
            pltpu.VMEM((128, 129), jnp.float32),
            pltpu.VMEM((128, 129), jnp.float32),
            pltpu.SemaphoreType.DMA,
            pltpu.SemaphoreType.DMA,
            pltpu.SemaphoreType.DMA,
            pltpu.SemaphoreType.DMA,
            pltpu.SemaphoreType.DMA,
            pltpu.SemaphoreType.DMA,
            pltpu.SemaphoreType.DMA,
            pltpu.SemaphoreType.DMA,
        ],
    )
    def gather_kernel(idx_hbm, table_hbm, out_hbm,
                      ia, ib, rows0, rows1, rows2, rows3, tiles0, tiles1,
                      sia, sib, sg0, sg1, sg2, sg3, ss0, ss1):
        wid = lax.axis_index("s") * 2 + lax.axis_index("c")
        g0 = wid * CPW
        idx0 = g0 * CB

        idx_bufs = (ia, ib)
        sems_i = (sia, sib)
        rows_bufs = (rows0, rows1, rows2, rows3)
        sems_g = (sg0, sg1, sg2, sg3)
        tiles_bufs = (tiles0, tiles1)
        sems_s = (ss0, ss1)

        def issue_super(s, q):
            off = pl.multiple_of(idx0 + s * SUP_IDX, SUP_IDX)
            pltpu.async_copy(
                idx_hbm.at[pl.ds(off, SUP_IDX)], idx_bufs[q], sems_i[q])

        def wait_super(q):
            pltpu.make_async_copy(
                idx_hbm.at[pl.ds(0, SUP_IDX)], idx_bufs[q], sems_i[q]).wait()

        def issue_gather(soff, q, p):
            pltpu.async_copy(
                table_hbm.at[idx_bufs[q].at[pl.ds(soff, CB)]],
                rows_bufs[p], sems_g[p])

        def wait_gather(p):
            pltpu.make_async_copy(
                table_hbm.at[pl.ds(0, CB)], rows_bufs[p], sems_g[p]).wait()

        def transpose(p, p2):
            rows = rows_bufs[p]
            tiles = tiles_bufs[p2]

            @plsc.parallel_loop(0, CB)
            def _(b):
                lane = lax.iota(jnp.int32, 16)
                bl = b >> 7
                ridx0 = bl * 32 + lane
                ridx1 = ridx0 + 16
                cidx = jnp.full((16,), 0, jnp.int32) + (b & 127)
                v0 = rows[b, pl.ds(0, 16)]
                plsc.store_scatter(tiles, [ridx0, cidx], v0)
                v1 = rows[b, pl.ds(16, 16)]
                plsc.store_scatter(tiles, [ridx1, cidx], v1)

        def issue_stores(g, p):
            h = g >> 5
            bb = (g & 31) * 4
            for dblk in range(4):
                for bl in range(4):
                    roff = pl.multiple_of(
                        ((h * 4 + dblk) * 128 + bb + bl) * 8, 8)
                    pltpu.async_copy(
                        tiles_bufs[p].at[
                            pl.ds(bl * 32 + dblk * 8, 8), pl.ds(0, 128)],
                        out_hbm.at[pl.ds(roff, 8)], sems_s[p])

        def wait_stores(p):
            pltpu.make_async_copy(
                out_hbm.at[pl.ds(0, 128)],
                tiles_bufs[p].at[:, pl.ds(0, 128)], sems_s[p]).wait()

        issue_super(0, 0)
        wait_super(0)
        for _p in range(4):
            issue_gather(_p * CB, 0, _p)
        pltpu.async_copy(out_hbm.at[pl.ds(0, 128)],
                         tiles0.at[:, pl.ds(0, 128)], ss0)
        pltpu.async_copy(out_hbm.at[pl.ds(0, 128)],
                         tiles1.at[:, pl.ds(0, 128)], ss1)

        def half(sp, hf, last_super=False):
            s = 2 * sp + hf
            q = hf
            qo = 1 - hf

            def jbody(j, carry):
                if not last_super:
                    @pl.when(j == 1)
                    def _():
                        issue_super(s + 1, qo)

                c0 = s * SUPER + 4 * j
                for p in range(4):
                    c = c0 + p
                    g = g0 + c
                    p2 = p % 2
                    wait_gather(p)
                    wait_stores(p2)
                    transpose(p, p2)
                    issue_stores(g, p2)
                    nxt = 4 * j + 4 + p
                    @pl.when(j < SUPER // 4 - 1)
                    def _():
                        issue_gather(nxt * CB, q, p)
                if not last_super:
                    @pl.when(j == SUPER // 4 - 1)
                    def _():
                        wait_super(qo)
                        for p in range(4):
                            issue_gather(p * CB, qo, p)
                return carry

            lax.fori_loop(0, SUPER // 4, jbody, 0)

        def spbody(sp, carry):
            half(sp, 0)
            half(sp, 1)
            return carry

        lax.fori_loop(0, N_SUP // 2 - 1, spbody, 0)
        half(N_SUP // 2 - 1, 0)
        half(N_SUP // 2 - 1, 1, last_super=True)

        wait_stores(0)
        wait_stores(1)

    return gather_kernel


_GATHER = _make_kernel()


def kernel(input_tokens, table):
    idx = input_tokens.T.reshape(B).astype(jnp.int32)
    r = _GATHER(idx, table)
    r5 = r.reshape(HIST, 4, BATCH // 128, 8, 128)
    t = r5.transpose(0, 1, 3, 2, 4)
    m = t.reshape(HIST, EMBED_DIM, BATCH)
    return m.transpose(2, 0, 1)


# speedup vs baseline: 1.3333x; 1.3333x over previous
import functools

import jax
import jax.numpy as jnp
from jax import lax
from jax.experimental import pallas as pl
from jax.experimental.pallas import tpu as pltpu
from jax.experimental.pallas import tpu_sc as plsc

NUM_TOKENS = 100000
EMBED_DIM = 32
BATCH = 16384
HIST = 200

B = BATCH * HIST
NW = 32
CB = 512
CH_PER_H = BATCH // CB
N_CH = B // CB
CPW = N_CH // NW
SUPER = 20
SUP_IDX = SUPER * CB
N_SUP = CPW // SUPER
TILE = 8 * 128
SEG = 4 * TILE
CHUNK_OUT = EMBED_DIM * CB
OUT_ELEMS = B * EMBED_DIM

assert CPW == N_SUP * SUPER and N_SUP % 2 == 0


def _make_kernel():
    mesh = plsc.VectorSubcoreMesh(core_axis_name="c", subcore_axis_name="s")

    @functools.partial(
        pl.kernel,
        out_type=jax.ShapeDtypeStruct((OUT_ELEMS // 128, 128), jnp.float32),
        mesh=mesh,
        compiler_params=pltpu.CompilerParams(
            use_tc_tiling_on_sc=False, needs_layout_passes=False),
        scratch_types=[
            pltpu.VMEM((SUP_IDX,), jnp.int32),
            pltpu.VMEM((SUP_IDX,), jnp.int32),
            pltpu.VMEM((CB, EMBED_DIM), jnp.float32),
            pltpu.VMEM((CB, EMBED_DIM), jnp.float32),
            pltpu.VMEM((128, 129), jnp.float32),
            pltpu.VMEM((128, 129), jnp.float32),
            pltpu.SemaphoreType.DMA,
            pltpu.SemaphoreType.DMA,
            pltpu.SemaphoreType.DMA,
            pltpu.SemaphoreType.DMA,
            pltpu.SemaphoreType.DMA,
            pltpu.SemaphoreType.DMA,
        ],
    )
    def gather_kernel(idx_hbm, table_hbm, out_hbm,
                      ia, ib, rows0, rows1, tiles0, tiles1,
                      sia, sib, sg0, sg1, ss0, ss1):
        wid = lax.axis_index("s") * 2 + lax.axis_index("c")
        g0 = wid * CPW
        idx0 = g0 * CB

        idx_bufs = (ia, ib)
        sems_i = (sia, sib)
        rows_bufs = (rows0, rows1)
        sems_g = (sg0, sg1)
        tiles_bufs = (tiles0, tiles1)
        sems_s = (ss0, ss1)

        def issue_super(s, q):
            off = pl.multiple_of(idx0 + s * SUP_IDX, SUP_IDX)
            pltpu.async_copy(
                idx_hbm.at[pl.ds(off, SUP_IDX)], idx_bufs[q], sems_i[q])

        def wait_super(q):
            pltpu.make_async_copy(
                idx_hbm.at[pl.ds(0, SUP_IDX)], idx_bufs[q], sems_i[q]).wait()

        def issue_gather(soff, q, p):
            pltpu.async_copy(
                table_hbm.at[idx_bufs[q].at[pl.ds(soff, CB)]],
                rows_bufs[p], sems_g[p])

        def wait_gather(p):
            pltpu.make_async_copy(
                table_hbm.at[pl.ds(0, CB)], rows_bufs[p], sems_g[p]).wait()

        def transpose(p):
            rows = rows_bufs[p]
            tiles = tiles_bufs[p]

            @plsc.parallel_loop(0, CB, unroll=4)
            def _(b):
                lane = lax.iota(jnp.int32, 16)
                bl = b >> 7
                ridx0 = bl * 32 + lane
                ridx1 = ridx0 + 16
                cidx = jnp.full((16,), 0, jnp.int32) + (b & 127)
                v0 = rows[b, pl.ds(0, 16)]
                plsc.store_scatter(tiles, [ridx0, cidx], v0)
                v1 = rows[b, pl.ds(16, 16)]
                plsc.store_scatter(tiles, [ridx1, cidx], v1)

        def issue_stores(g, p):
            h = g >> 5
            bb = (g & 31) * 4
            for dblk in range(4):
                for bl in range(4):
                    roff = pl.multiple_of(
                        ((h * 4 + dblk) * 128 + bb + bl) * 8, 8)
                    pltpu.async_copy(
                        tiles_bufs[p].at[
                            pl.ds(bl * 32 + dblk * 8, 8), pl.ds(0, 128)],
                        out_hbm.at[pl.ds(roff, 8)], sems_s[p])

        def wait_stores(p):
            pltpu.make_async_copy(
                out_hbm.at[pl.ds(0, 128)],
                tiles_bufs[p].at[:, pl.ds(0, 128)], sems_s[p]).wait()

        issue_super(0, 0)
        wait_super(0)
        issue_gather(0, 0, 0)
        issue_gather(CB, 0, 1)
        pltpu.async_copy(out_hbm.at[pl.ds(0, 128)],
                         tiles0.at[:, pl.ds(0, 128)], ss0)
        pltpu.async_copy(out_hbm.at[pl.ds(0, 128)],
                         tiles1.at[:, pl.ds(0, 128)], ss1)

        def half(sp, hf, last_super=False):
            s = 2 * sp + hf
            q = hf
            qo = 1 - hf

            def jbody(j, carry):
                if not last_super:
                    @pl.when(j == 1)
                    def _():
                        issue_super(s + 1, qo)

                c0 = s * SUPER + 2 * j
                for p in range(2):
                    c = c0 + p
                    g = g0 + c
                    wait_gather(p)
                    wait_stores(p)
                    transpose(p)
                    issue_stores(g, p)
                    nxt = 2 * j + 2 + p
                    @pl.when(j < SUPER // 2 - 1)
                    def _():
                        issue_gather(nxt * CB, q, p)
                if not last_super:
                    @pl.when(j == SUPER // 2 - 1)
                    def _():
                        wait_super(qo)
                        issue_gather(0, qo, 0)
                        issue_gather(CB, qo, 1)
                return carry

            lax.fori_loop(0, SUPER // 2, jbody, 0)

        def spbody(sp, carry):
            half(sp, 0)
            half(sp, 1)
            return carry

        lax.fori_loop(0, N_SUP // 2 - 1, spbody, 0)
        half(N_SUP // 2 - 1, 0)
        half(N_SUP // 2 - 1, 1, last_super=True)

        wait_stores(0)
        wait_stores(1)

    return gather_kernel


_GATHER = _make_kernel()


def kernel(input_tokens, table):
    idx = input_tokens.T.reshape(B).astype(jnp.int32)
    r = _GATHER(idx, table)
    r5 = r.reshape(HIST, 4, BATCH // 128, 8, 128)
    t = r5.transpose(0, 1, 3, 2, 4)
    m = t.reshape(HIST, EMBED_DIM, BATCH)
    return m.transpose(2, 0, 1)


# speedup vs baseline: 1.3397x; 1.0048x over previous
import functools

import jax
import jax.numpy as jnp
from jax import lax
from jax.experimental import pallas as pl
from jax.experimental.pallas import tpu as pltpu
from jax.experimental.pallas import tpu_sc as plsc

NUM_TOKENS = 100000
EMBED_DIM = 32
BATCH = 16384
HIST = 200

B = BATCH * HIST
NW = 32
CB = 512
CH_PER_H = BATCH // CB
N_CH = B // CB
CPW = N_CH // NW
SUPER = 20
SUP_IDX = SUPER * CB
N_SUP = CPW // SUPER
TILE = 8 * 128
SEG = 4 * TILE
CHUNK_OUT = EMBED_DIM * CB
OUT_ELEMS = B * EMBED_DIM

assert CPW == N_SUP * SUPER and N_SUP % 2 == 0


def _make_kernel():
    mesh = plsc.VectorSubcoreMesh(core_axis_name="c", subcore_axis_name="s")

    @functools.partial(
        pl.kernel,
        out_type=jax.ShapeDtypeStruct((OUT_ELEMS // 128, 128), jnp.float32),
        mesh=mesh,
        compiler_params=pltpu.CompilerParams(
            use_tc_tiling_on_sc=False, needs_layout_passes=False),
        scratch_types=[
            pltpu.VMEM((SUP_IDX,), jnp.int32),
            pltpu.VMEM((SUP_IDX,), jnp.int32),
            pltpu.VMEM((CB, EMBED_DIM), jnp.float32),
            pltpu.VMEM((CB, EMBED_DIM), jnp.float32),
            pltpu.VMEM((128, 129), jnp.float32),
            pltpu.VMEM((128, 129), jnp.float32),
            pltpu.SemaphoreType.DMA,
            pltpu.SemaphoreType.DMA,
            pltpu.SemaphoreType.DMA,
            pltpu.SemaphoreType.DMA,
            pltpu.SemaphoreType.DMA,
            pltpu.SemaphoreType.DMA,
        ],
    )
    def gather_kernel(idx_hbm, table_hbm, out_hbm,
                      ia, ib, rows0, rows1, tiles0, tiles1,
                      sia, sib, sg0, sg1, ss0, ss1):
        wid = lax.axis_index("s") * 2 + lax.axis_index("c")
        g0 = wid * CPW
        idx0 = g0 * CB

        idx_bufs = (ia, ib)
        sems_i = (sia, sib)
        rows_bufs = (rows0, rows1)
        sems_g = (sg0, sg1)
        tiles_bufs = (tiles0, tiles1)
        sems_s = (ss0, ss1)

        def issue_super(s, q):
            off = pl.multiple_of(idx0 + s * SUP_IDX, SUP_IDX)
            pltpu.async_copy(
                idx_hbm.at[pl.ds(off, SUP_IDX)], idx_bufs[q], sems_i[q])

        def wait_super(q):
            pltpu.make_async_copy(
                idx_hbm.at[pl.ds(0, SUP_IDX)], idx_bufs[q], sems_i[q]).wait()

        def issue_gather(soff, q, p):
            pltpu.async_copy(
                table_hbm.at[idx_bufs[q].at[pl.ds(soff, CB)]],
                rows_bufs[p], sems_g[p])

        def wait_gather(p):
            pltpu.make_async_copy(
                table_hbm.at[pl.ds(0, CB)], rows_bufs[p], sems_g[p]).wait()

        def transpose(p):
            rows = rows_bufs[p]
            tiles = tiles_bufs[p]

            @plsc.parallel_loop(0, CB, unroll=8)
            def _(b):
                lane = lax.iota(jnp.int32, 16)
                bl = b >> 7
                ridx0 = bl * 32 + lane
                ridx1 = ridx0 + 16
                cidx = jnp.full((16,), 0, jnp.int32) + (b & 127)
                v0 = rows[b, pl.ds(0, 16)]
                plsc.store_scatter(tiles, [ridx0, cidx], v0)
                v1 = rows[b, pl.ds(16, 16)]
                plsc.store_scatter(tiles, [ridx1, cidx], v1)

        def issue_stores(g, p):
            h = g >> 5
            bb = (g & 31) * 4
            for dblk in range(4):
                for bl in range(4):
                    roff = pl.multiple_of(
                        ((h * 4 + dblk) * 128 + bb + bl) * 8, 8)
                    pltpu.async_copy(
                        tiles_bufs[p].at[
                            pl.ds(bl * 32 + dblk * 8, 8), pl.ds(0, 128)],
                        out_hbm.at[pl.ds(roff, 8)], sems_s[p])

        def wait_stores(p):
            pltpu.make_async_copy(
                out_hbm.at[pl.ds(0, 128)],
                tiles_bufs[p].at[:, pl.ds(0, 128)], sems_s[p]).wait()

        issue_super(0, 0)
        wait_super(0)
        issue_gather(0, 0, 0)
        issue_gather(CB, 0, 1)
        pltpu.async_copy(out_hbm.at[pl.ds(0, 128)],
                         tiles0.at[:, pl.ds(0, 128)], ss0)
        pltpu.async_copy(out_hbm.at[pl.ds(0, 128)],
                         tiles1.at[:, pl.ds(0, 128)], ss1)

        def half(sp, hf, last_super=False):
            s = 2 * sp + hf
            q = hf
            qo = 1 - hf

            def jbody(j, carry):
                if not last_super:
                    @pl.when(j == 1)
                    def _():
                        issue_super(s + 1, qo)

                c0 = s * SUPER + 2 * j
                for p in range(2):
                    c = c0 + p
                    g = g0 + c
                    wait_gather(p)
                    wait_stores(p)
                    transpose(p)
                    issue_stores(g, p)
                    nxt = 2 * j + 2 + p
                    @pl.when(j < SUPER // 2 - 1)
                    def _():
                        issue_gather(nxt * CB, q, p)
                if not last_super:
                    @pl.when(j == SUPER // 2 - 1)
                    def _():
                        wait_super(qo)
                        issue_gather(0, qo, 0)
                        issue_gather(CB, qo, 1)
                return carry

            lax.fori_loop(0, SUPER // 2, jbody, 0)

        def spbody(sp, carry):
            half(sp, 0)
            half(sp, 1)
            return carry

        lax.fori_loop(0, N_SUP // 2 - 1, spbody, 0)
        half(N_SUP // 2 - 1, 0)
        half(N_SUP // 2 - 1, 1, last_super=True)

        wait_stores(0)
        wait_stores(1)

    return gather_kernel


_GATHER = _make_kernel()


def kernel(input_tokens, table):
    idx = input_tokens.T.reshape(B).astype(jnp.int32)
    r = _GATHER(idx, table)
    r5 = r.reshape(HIST, 4, BATCH // 128, 8, 128)
    t = r5.transpose(0, 1, 3, 2, 4)
    m = t.reshape(HIST, EMBED_DIM, BATCH)
    return m.transpose(2, 0, 1)
